# baseline (speedup 1.0000x reference)
import functools
import jax
import jax.numpy as jnp
from jax import lax
from jax.experimental import pallas as pl
from jax.experimental.pallas import tpu as pltpu
from jax.experimental.pallas import tpu_sc as plsc

VOCAB = 151936
DIM = 1152
B = 8
S = 2048
NTOK = B * S


@functools.lru_cache(maxsize=None)
def _build_gather():
    info = plsc.get_sparse_core_info()
    nc, ns = info.num_cores, info.num_subcores
    nw = nc * ns
    rows_per_w = NTOK // nw
    chunk = 16
    nbuf = 6
    nchunk = rows_per_w // chunk

    mesh = plsc.VectorSubcoreMesh(core_axis_name="c", subcore_axis_name="s")

    @functools.partial(
        pl.kernel,
        mesh=mesh,
        out_type=jax.ShapeDtypeStruct((NTOK, DIM), jnp.float32),
        scratch_types=[
            pltpu.VMEM((rows_per_w,), jnp.int32),
        ]
        + [pltpu.VMEM((chunk, DIM), jnp.float32) for _ in range(nbuf)]
        + [pltpu.SemaphoreType.DMA for _ in range(2 * nbuf)],
    )
    def gather(idx_hbm, table_hbm, out_hbm, idx_v, *bufs_and_sems):
        bufs = bufs_and_sems[:nbuf]
        sem_g = bufs_and_sems[nbuf:2 * nbuf]
        sem_o = bufs_and_sems[2 * nbuf:]
        wid = lax.axis_index("s") * nc + lax.axis_index("c")
        base = wid * rows_per_w
        pltpu.sync_copy(idx_hbm.at[pl.ds(base, rows_per_w)], idx_v)

        def start_gather(g):
            p = g % nbuf
            pltpu.async_copy(
                table_hbm.at[idx_v.at[pl.ds(g * chunk, chunk)]],
                bufs[p], sem_g[p])

        def start_out(g):
            p = g % nbuf
            pltpu.async_copy(
                bufs[p], out_hbm.at[pl.ds(base + g * chunk, chunk)],
                sem_o[p])

        def wait_gather(g):
            p = g % nbuf
            pltpu.make_async_copy(
                table_hbm.at[idx_v.at[pl.ds(g * chunk, chunk)]],
                bufs[p], sem_g[p]).wait()

        def wait_out(g):
            p = g % nbuf
            pltpu.make_async_copy(
                bufs[p], out_hbm.at[pl.ds(base + g * chunk, chunk)],
                sem_o[p]).wait()

        depth = nbuf - 1
        outs_waited = set()
        for g0 in range(min(depth, nchunk)):
            start_gather(g0)
        for g in range(nchunk):
            nxt = g + depth
            if nxt < nchunk:
                prev_out = nxt - nbuf
                if prev_out >= 0:
                    wait_out(prev_out)
                    outs_waited.add(prev_out)
                start_gather(nxt)
            wait_gather(g)
            start_out(g)
        for g in range(nchunk):
            if g not in outs_waited:
                wait_out(g)

    return gather


def kernel(x, emb_weight):
    idx = x.reshape(NTOK).astype(jnp.int32)
    out = _build_gather()(idx, emb_weight)
    return out.reshape(B, S, DIM)
